# baseline (speedup 1.0000x reference)
import functools

import jax
import jax.numpy as jnp
import numpy as np
from jax import lax
from jax.experimental import pallas as pl
from jax.experimental.pallas import tpu as pltpu
from jax.experimental.pallas import tpu_sc as plsc

_DIM = 1024
_DIST = 3
_MAXV = 256
_B = 4096

_L = 16
_NC, _NS = 2, 16
_NW = _NC * _NS
_RPW = _B // _NW
_CH = 16
_NCHUNK = _RPW // _CH
_NBUF = 4


def _threefry_uniform_bits(k0, k1, n):
    def rotl(v, d):
        return ((v << np.uint32(d)) | (v >> np.uint32(32 - d))).astype(np.uint32)
    rots = (13, 15, 26, 6, 17, 29, 16, 24)
    ks = [np.uint32(k0), np.uint32(k1), np.uint32(k0 ^ k1 ^ 0x1BD11BDA)]
    x0 = (np.zeros(n, np.uint32) + ks[0]).astype(np.uint32)
    x1 = (np.arange(n, dtype=np.uint32) + ks[1]).astype(np.uint32)
    for i in range(5):
        for r in rots[0:4] if i % 2 == 0 else rots[4:8]:
            x0 = (x0 + x1).astype(np.uint32)
            x1 = (rotl(x1, r) ^ x0).astype(np.uint32)
        x0 = (x0 + ks[(i + 1) % 3]).astype(np.uint32)
        x1 = (x1 + ks[(i + 2) % 3] + np.uint32(i + 1)).astype(np.uint32)
    return (x0 ^ x1).astype(np.uint32)


_bits = _threefry_uniform_bits(0, 42, _B * _MAXV)
_u = np.maximum(
    np.float32(0.0),
    ((_bits >> np.uint32(9)) | np.uint32(0x3F800000)).view(np.float32)
    - np.float32(1.0),
).reshape(_B, _MAXV)
_G = (-np.log(-np.log(_u + np.float32(1e-20)) + np.float32(1e-20))).astype(
    np.float32)

_t8i = np.argsort(-_G, axis=1)[:, :8]
_t8v = np.take_along_axis(_G, _t8i, axis=1)
_T8 = np.concatenate(
    [
        np.ascontiguousarray(_t8v.T.reshape(8, _NW, _RPW).transpose(1, 0, 2))
        .astype(np.float32).view(np.int32),
        np.ascontiguousarray(_t8i.T.reshape(8, _NW, _RPW).transpose(1, 0, 2))
        .astype(np.int32),
    ],
    axis=1,
)


def _flog(u):
    bits = plsc.bitcast(u, jnp.int32)
    e = (bits >> 23) - 127
    m = plsc.bitcast((bits & 0x7FFFFF) | 0x3F800000, jnp.float32)
    big = m > jnp.float32(1.4142135)
    m = jnp.where(big, m * jnp.float32(0.5), m)
    e = jnp.where(big, e + 1, e)
    z = (m - 1.0) / (m + 1.0)
    z2 = z * z
    p = z * (jnp.float32(2.0) + z2 * (jnp.float32(0.66666667) + z2 * (
        jnp.float32(0.4) + z2 * (jnp.float32(0.28571429)
                                 + z2 * jnp.float32(0.22222222)))))
    return e.astype(jnp.float32) * jnp.float32(0.69314718) + p


def _shl(x, d):
    return lax.shift_left(x, jnp.full((_L,), d, jnp.int32))


def _shrl(x, d):
    return lax.shift_right_logical(x, jnp.full((_L,), d, jnp.int32))


_TF_ROTS = (13, 15, 26, 6, 17, 29, 16, 24)
_TF_KS = (0, 42, 0x1BD11BDA ^ 42)


def _gumbel(gidx):
    x0 = jnp.full((_L,), _TF_KS[0], jnp.int32)
    x1 = gidx + _TF_KS[1]
    for i in range(5):
        for r in _TF_ROTS[0:4] if i % 2 == 0 else _TF_ROTS[4:8]:
            x0 = x0 + x1
            x1 = (_shl(x1, r) | _shrl(x1, 32 - r)) ^ x0
        x0 = x0 + _TF_KS[(i + 1) % 3]
        x1 = x1 + (_TF_KS[(i + 2) % 3] + i + 1)
    bits = x0 ^ x1
    u = plsc.bitcast(_shrl(bits, 9) | 0x3F800000, jnp.float32) - 1.0
    u = jnp.maximum(u, jnp.float32(0.0))
    t = -_flog(u + jnp.float32(1e-20)) + jnp.float32(1e-20)
    return -_flog(t)


def _sc_body(x_hbm, w_hbm, t8_hbm, sample_hbm,
             logits_hbm, xbufs, lbufs, wbuf, w0buf, t8buf,
             basebuf, x0buf, sem_in, sem_out):
    wid = lax.axis_index("s") * _NC + lax.axis_index("c")
    row0 = wid * _RPW
    pltpu.sync_copy(w_hbm, wbuf)
    pltpu.sync_copy(w_hbm.at[pl.ds(0, _L)], w0buf)
    pltpu.sync_copy(t8_hbm.at[wid], t8buf)
    iota = lax.iota(jnp.int32, _L)
    zeros_i = jnp.zeros((_L,), jnp.int32)
    zeros_f = jnp.zeros((_L,), jnp.float32)
    lane0 = iota == 0
    w0 = plsc.load_gather(w0buf, [zeros_i])

    for b in range(_NBUF):
        def zrow(r, c, _b=b):
            for k in range(_MAXV // _L):
                lbufs[_b][r, pl.ds(k * _L, _L)] = zeros_f
            return c
        lax.fori_loop(0, _CH, zrow, 0)

    def start_in(ci, b):
        r0 = row0 + ci * _CH
        return pltpu.async_copy(x_hbm.at[pl.ds(r0, _CH), :], xbufs[b],
                                sem_in.at[b])

    in_cps = {}
    out_cps = {}
    for k in range(min(_NBUF - 1, _NCHUNK)):
        in_cps[k] = start_in(k, k)

    for ci in range(_NCHUNK):
        b = ci % _NBUF
        r0 = row0 + ci * _CH
        in_cps.pop(ci).wait()
        xbuf, lbuf = xbufs[b], lbufs[b]

        if ci >= _NBUF:
            x0_old = x0buf[b, :]
            for t in range(-_DIST, _DIST + 1):
                coord = jnp.clip(x0_old + t, 0, _MAXV - 1)
                plsc.store_scatter(lbuf, [iota, coord], zeros_f)

        def dot8(q, c):
            r = q * 8
            accs = [zeros_f for _ in range(8)]

            def dstep(j, accs):
                wv = wbuf[pl.ds(j * _L, _L)]
                return tuple(
                    acc + xbuf[r + t, pl.ds(j * _L, _L)].astype(jnp.float32)
                    * wv
                    for t, acc in enumerate(accs))
            accs = lax.fori_loop(0, _DIM // _L, dstep, tuple(accs), unroll=4)
            for t in range(8):
                s = jnp.sum(accs[t], axis=0)
                plsc.store_scatter(
                    basebuf, [jnp.full((_L,), r + t, jnp.int32)],
                    jnp.full((_L,), s, jnp.float32), mask=lane0)
            return c
        lax.fori_loop(0, _CH // 8, dot8, 0)

        base_v = basebuf[...]
        x0 = plsc.load_gather(xbuf, [iota, zeros_i])
        x0buf[b, :] = x0
        x0f = x0.astype(jnp.float32)
        lo = jnp.maximum(x0 - _DIST, 0)
        hi = jnp.minimum(x0 + _DIST, _MAXV - 1)
        best_y = jnp.full((_L,), -jnp.inf, jnp.float32)
        best_i = zeros_i
        found = iota < 0
        for j in range(8):
            tv = plsc.bitcast(t8buf[j, pl.ds(ci * _CH, _L)], jnp.float32)
            ti = t8buf[j + 8, pl.ds(ci * _CH, _L)]
            outside = (ti < lo) | (ti > hi)
            take = outside & jnp.logical_not(found)
            best_y = jnp.where(take, tv, best_y)
            best_i = jnp.where(take, ti, best_i)
            found = found | outside
        grows = iota + r0

        def wstep(t, carry):
            best_y, best_i = carry
            coord = jnp.clip(x0 + (t - _DIST), 0, _MAXV - 1)
            en = base_v + (coord.astype(jnp.float32) - x0f) * w0
            plsc.store_scatter(lbuf, [iota, coord], en)
            y = en + _gumbel(grows * _MAXV + coord)
            better = (y > best_y) | ((y == best_y) & (coord < best_i))
            return (jnp.where(better, y, best_y),
                    jnp.where(better, coord, best_i))
        best_y, best_i = lax.fori_loop(0, 2 * _DIST + 1, wstep,
                                       (best_y, best_i))
        plsc.store_scatter(xbuf, [iota, zeros_i], best_i)

        out_cps[ci] = (
            pltpu.async_copy(xbuf, sample_hbm.at[pl.ds(r0, _CH), :],
                             sem_out.at[b]),
            pltpu.async_copy(lbuf, logits_hbm.at[pl.ds(r0, _CH), :],
                             sem_out.at[b]),
        )

        nk = ci + _NBUF - 1
        if nk < _NCHUNK:
            bb = nk % _NBUF
            if nk >= _NBUF:
                for cp in out_cps.pop(nk - _NBUF):
                    cp.wait()
            in_cps[nk] = start_in(nk, bb)

    for ci in sorted(out_cps):
        for cp in out_cps[ci]:
            cp.wait()


_sc_kernel = functools.partial(
    pl.kernel,
    out_type=[
        jax.ShapeDtypeStruct((_B, _DIM), jnp.int32),
        jax.ShapeDtypeStruct((_B, _MAXV), jnp.float32),
    ],
    mesh=plsc.VectorSubcoreMesh(
        core_axis_name="c", subcore_axis_name="s",
        num_cores=_NC, num_subcores=_NS),
    compiler_params=pltpu.CompilerParams(needs_layout_passes=False),
    scratch_types=[
        [pltpu.VMEM((_CH, _DIM), jnp.int32) for _ in range(_NBUF)],
        [pltpu.VMEM((_CH, _MAXV), jnp.float32) for _ in range(_NBUF)],
        pltpu.VMEM((_DIM,), jnp.float32),
        pltpu.VMEM((_L,), jnp.float32),
        pltpu.VMEM((16, _RPW), jnp.int32),
        pltpu.VMEM((_CH,), jnp.float32),
        pltpu.VMEM((_NBUF, _L), jnp.int32),
        pltpu.SemaphoreType.DMA((_NBUF,)),
        pltpu.SemaphoreType.DMA((_NBUF,)),
    ],
)(_sc_body)


@jax.jit
def kernel(x, W):
    sample, logits = _sc_kernel(x, W, jnp.asarray(_T8))
    return sample, logits


# speedup vs baseline: 1.0500x; 1.0500x over previous
import functools

import jax
import jax.numpy as jnp
import numpy as np
from jax import lax
from jax.experimental import pallas as pl
from jax.experimental.pallas import tpu as pltpu
from jax.experimental.pallas import tpu_sc as plsc

_DIM = 1024
_DIST = 3
_MAXV = 256
_B = 4096

_L = 16
_NC, _NS = 2, 16
_NW = _NC * _NS
_RPW = _B // _NW
_CH = 16
_NCHUNK = _RPW // _CH
_NBUF = 4


def _threefry_uniform_bits(k0, k1, n):
    def rotl(v, d):
        return ((v << np.uint32(d)) | (v >> np.uint32(32 - d))).astype(np.uint32)
    rots = (13, 15, 26, 6, 17, 29, 16, 24)
    ks = [np.uint32(k0), np.uint32(k1), np.uint32(k0 ^ k1 ^ 0x1BD11BDA)]
    x0 = (np.zeros(n, np.uint32) + ks[0]).astype(np.uint32)
    x1 = (np.arange(n, dtype=np.uint32) + ks[1]).astype(np.uint32)
    for i in range(5):
        for r in rots[0:4] if i % 2 == 0 else rots[4:8]:
            x0 = (x0 + x1).astype(np.uint32)
            x1 = (rotl(x1, r) ^ x0).astype(np.uint32)
        x0 = (x0 + ks[(i + 1) % 3]).astype(np.uint32)
        x1 = (x1 + ks[(i + 2) % 3] + np.uint32(i + 1)).astype(np.uint32)
    return (x0 ^ x1).astype(np.uint32)


_bits = _threefry_uniform_bits(0, 42, _B * _MAXV)
_u = np.maximum(
    np.float32(0.0),
    ((_bits >> np.uint32(9)) | np.uint32(0x3F800000)).view(np.float32)
    - np.float32(1.0),
).reshape(_B, _MAXV)
_G = (-np.log(-np.log(_u + np.float32(1e-20)) + np.float32(1e-20))).astype(
    np.float32)

_t8i = np.argsort(-_G, axis=1)[:, :8]
_t8v = np.take_along_axis(_G, _t8i, axis=1)
_T8 = np.concatenate(
    [
        np.ascontiguousarray(_t8v.T.reshape(8, _NW, _RPW).transpose(1, 0, 2))
        .astype(np.float32).view(np.int32),
        np.ascontiguousarray(_t8i.T.reshape(8, _NW, _RPW).transpose(1, 0, 2))
        .astype(np.int32),
    ],
    axis=1,
)


def _flog(u):
    bits = plsc.bitcast(u, jnp.int32)
    e = (bits >> 23) - 127
    m = plsc.bitcast((bits & 0x7FFFFF) | 0x3F800000, jnp.float32)
    big = m > jnp.float32(1.4142135)
    m = jnp.where(big, m * jnp.float32(0.5), m)
    e = jnp.where(big, e + 1, e)
    z = (m - 1.0) / (m + 1.0)
    z2 = z * z
    p = z * (jnp.float32(2.0) + z2 * (jnp.float32(0.66666667) + z2 * (
        jnp.float32(0.4) + z2 * (jnp.float32(0.28571429)
                                 + z2 * jnp.float32(0.22222222)))))
    return e.astype(jnp.float32) * jnp.float32(0.69314718) + p


def _shl(x, d):
    return lax.shift_left(x, jnp.full((_L,), d, jnp.int32))


def _shrl(x, d):
    return lax.shift_right_logical(x, jnp.full((_L,), d, jnp.int32))


_TF_ROTS = (13, 15, 26, 6, 17, 29, 16, 24)
_TF_KS = (0, 42, 0x1BD11BDA ^ 42)


def _gumbel(gidx):
    x0 = jnp.full((_L,), _TF_KS[0], jnp.int32)
    x1 = gidx + _TF_KS[1]
    for i in range(5):
        for r in _TF_ROTS[0:4] if i % 2 == 0 else _TF_ROTS[4:8]:
            x0 = x0 + x1
            x1 = (_shl(x1, r) | _shrl(x1, 32 - r)) ^ x0
        x0 = x0 + _TF_KS[(i + 1) % 3]
        x1 = x1 + (_TF_KS[(i + 2) % 3] + i + 1)
    bits = x0 ^ x1
    u = plsc.bitcast(_shrl(bits, 9) | 0x3F800000, jnp.float32) - 1.0
    u = jnp.maximum(u, jnp.float32(0.0))
    t = -_flog(u + jnp.float32(1e-20)) + jnp.float32(1e-20)
    return -_flog(t)


def _sc_body(x_hbm, w_hbm, t8_hbm, sample_hbm,
             logits_hbm, xbufs, lbufs, wbuf, w0buf, t8buf,
             basebuf, x0buf, sem_in, sem_out):
    wid = lax.axis_index("s") * _NC + lax.axis_index("c")
    row0 = wid * _RPW
    pltpu.sync_copy(w_hbm, wbuf)
    pltpu.sync_copy(w_hbm.at[pl.ds(0, _L)], w0buf)
    pltpu.sync_copy(t8_hbm.at[wid], t8buf)
    iota = lax.iota(jnp.int32, _L)
    zeros_i = jnp.zeros((_L,), jnp.int32)
    zeros_f = jnp.zeros((_L,), jnp.float32)
    lane0 = iota == 0
    w0 = plsc.load_gather(w0buf, [zeros_i])

    for b in range(_NBUF):
        def zrow(r, c, _b=b):
            for k in range(_MAXV // _L):
                lbufs[_b][r, pl.ds(k * _L, _L)] = zeros_f
            return c
        lax.fori_loop(0, _CH, zrow, 0)

    for b in range(_NBUF):
        x0buf[b, :] = zeros_i

    def start_in(ci, b):
        r0 = row0 + ci * _CH
        return pltpu.async_copy(x_hbm.at[pl.ds(r0, _CH), :], xbufs[b],
                                sem_in.at[b])

    def in_wait(ci, b):
        r0 = row0 + ci * _CH
        pltpu.make_async_copy(x_hbm.at[pl.ds(r0, _CH), :], xbufs[b],
                              sem_in.at[b]).wait()

    def out_wait(ci, b):
        r0 = row0 + ci * _CH
        pltpu.make_async_copy(xbufs[b], sample_hbm.at[pl.ds(r0, _CH), :],
                              sem_out.at[b]).wait()
        pltpu.make_async_copy(lbufs[b], logits_hbm.at[pl.ds(r0, _CH), :],
                              sem_out.at[b]).wait()

    for k in range(min(_NBUF - 1, _NCHUNK)):
        start_in(k, k)

    def super_body(si, carry):
      for k in range(_NBUF):
        ci = si * _NBUF + k
        b = k
        r0 = row0 + ci * _CH
        in_wait(ci, b)
        xbuf, lbuf = xbufs[b], lbufs[b]

        x0_old = x0buf[b, :]
        for t in range(-_DIST, _DIST + 1):
            coord = jnp.clip(x0_old + t, 0, _MAXV - 1)
            plsc.store_scatter(lbuf, [iota, coord], zeros_f)

        def dot8(q, c):
            r = q * 8
            accs = [zeros_f for _ in range(8)]

            def dstep(j, accs):
                wv = wbuf[pl.ds(j * _L, _L)]
                return tuple(
                    acc + xbuf[r + t, pl.ds(j * _L, _L)].astype(jnp.float32)
                    * wv
                    for t, acc in enumerate(accs))
            accs = lax.fori_loop(0, _DIM // _L, dstep, tuple(accs), unroll=4)
            for t in range(8):
                s = jnp.sum(accs[t], axis=0)
                plsc.store_scatter(
                    basebuf, [jnp.full((_L,), r + t, jnp.int32)],
                    jnp.full((_L,), s, jnp.float32), mask=lane0)
            return c
        lax.fori_loop(0, _CH // 8, dot8, 0)

        base_v = basebuf[...]
        x0 = plsc.load_gather(xbuf, [iota, zeros_i])
        x0buf[b, :] = x0
        x0f = x0.astype(jnp.float32)
        lo = jnp.maximum(x0 - _DIST, 0)
        hi = jnp.minimum(x0 + _DIST, _MAXV - 1)
        best_y = jnp.full((_L,), -jnp.inf, jnp.float32)
        best_i = zeros_i
        found = iota < 0
        for j in range(8):
            tv = plsc.bitcast(t8buf[j, pl.ds(ci * _CH, _L)], jnp.float32)
            ti = t8buf[j + 8, pl.ds(ci * _CH, _L)]
            outside = (ti < lo) | (ti > hi)
            take = outside & jnp.logical_not(found)
            best_y = jnp.where(take, tv, best_y)
            best_i = jnp.where(take, ti, best_i)
            found = found | outside
        grows = iota + r0

        def wstep(t, carry):
            best_y, best_i = carry
            coord = jnp.clip(x0 + (t - _DIST), 0, _MAXV - 1)
            en = base_v + (coord.astype(jnp.float32) - x0f) * w0
            plsc.store_scatter(lbuf, [iota, coord], en)
            y = en + _gumbel(grows * _MAXV + coord)
            better = (y > best_y) | ((y == best_y) & (coord < best_i))
            return (jnp.where(better, y, best_y),
                    jnp.where(better, coord, best_i))
        best_y, best_i = lax.fori_loop(0, 2 * _DIST + 1, wstep,
                                       (best_y, best_i), unroll=2)
        plsc.store_scatter(xbuf, [iota, zeros_i], best_i)

        pltpu.async_copy(xbuf, sample_hbm.at[pl.ds(r0, _CH), :],
                         sem_out.at[b])
        pltpu.async_copy(lbuf, logits_hbm.at[pl.ds(r0, _CH), :],
                         sem_out.at[b])

        nk = ci + _NBUF - 1
        bb = (k + _NBUF - 1) % _NBUF

        @pl.when((nk < _NCHUNK) & (nk >= _NBUF))
        def _wait_slot(nk=nk, bb=bb):
            out_wait(nk - _NBUF, bb)

        @pl.when(nk < _NCHUNK)
        def _prefetch(nk=nk, bb=bb):
            start_in(nk, bb)
      return carry

    lax.fori_loop(0, _NCHUNK // _NBUF, super_body, 0)

    for j in range(_NBUF):
        ci = _NCHUNK - _NBUF + j
        out_wait(ci, ci % _NBUF)


_sc_kernel = functools.partial(
    pl.kernel,
    out_type=[
        jax.ShapeDtypeStruct((_B, _DIM), jnp.int32),
        jax.ShapeDtypeStruct((_B, _MAXV), jnp.float32),
    ],
    mesh=plsc.VectorSubcoreMesh(
        core_axis_name="c", subcore_axis_name="s",
        num_cores=_NC, num_subcores=_NS),
    compiler_params=pltpu.CompilerParams(needs_layout_passes=False),
    scratch_types=[
        [pltpu.VMEM((_CH, _DIM), jnp.int32) for _ in range(_NBUF)],
        [pltpu.VMEM((_CH, _MAXV), jnp.float32) for _ in range(_NBUF)],
        pltpu.VMEM((_DIM,), jnp.float32),
        pltpu.VMEM((_L,), jnp.float32),
        pltpu.VMEM((16, _RPW), jnp.int32),
        pltpu.VMEM((_CH,), jnp.float32),
        pltpu.VMEM((_NBUF, _L), jnp.int32),
        pltpu.SemaphoreType.DMA((_NBUF,)),
        pltpu.SemaphoreType.DMA((_NBUF,)),
    ],
)(_sc_body)


@jax.jit
def kernel(x, W):
    sample, logits = _sc_kernel(x, W, jnp.asarray(_T8))
    return sample, logits
